# baseline (speedup 1.0000x reference)
import jax
import jax.numpy as jnp
from jax.experimental import pallas as pl
from jax.experimental.pallas import tpu as pltpu

_N = 1_000_000
_BN = 65536
_GRID = -(-_N // _BN)


def _body(cs_ref, x_ref, m_ref, o_ref):
    c = cs_ref[0]
    s = cs_ref[1]
    x = x_ref[...]
    m = m_ref[...].reshape(1, _BN)
    top = x[0:3]
    mid = x[3:6]
    rt = c * top - s * mid
    rm = s * top + c * mid
    o_ref[...] = jnp.concatenate(
        [top + m * (rt - top), mid + m * (rm - mid), x[6:8]], axis=0)


@jax.jit
def _rotate(Rt8, mvec, cs):
    return pl.pallas_call(
        _body,
        grid=(_GRID,),
        in_specs=[
            pl.BlockSpec(memory_space=pltpu.MemorySpace.SMEM),
            pl.BlockSpec((8, _BN), lambda i: (0, i)),
            pl.BlockSpec((_BN,), lambda i: (i,)),
        ],
        out_specs=pl.BlockSpec((8, _BN), lambda i: (0, i)),
        out_shape=jax.ShapeDtypeStruct((8, _N), jnp.float32),
    )(cs, Rt8, mvec)


def kernel(R, mask, value):
    angle = jnp.float32(value)
    cs = jnp.stack([jnp.cos(angle), jnp.sin(angle)])
    Rt8 = jnp.transpose(R, (1, 2, 0)).reshape(9, _N)[0:8]
    mvec = mask.astype(jnp.float32)
    out8 = _rotate(Rt8, mvec, cs)
    row8 = R[:, 2, 2].reshape(1, _N)
    outT = jnp.concatenate([out8, row8], axis=0)
    return jnp.transpose(outT.reshape(3, 3, _N), (2, 0, 1))


# speedup vs baseline: 1.1218x; 1.1218x over previous
import jax
import jax.numpy as jnp
from jax.experimental import pallas as pl
from jax.experimental.pallas import tpu as pltpu

_N = 1_000_000
_BN = 65536
_GRID = -(-_N // _BN)


def _body(cs_ref, x_ref, m_ref, o_ref):
    c = cs_ref[0]
    s = cs_ref[1]
    x = x_ref[...]
    m = m_ref[...]
    top = x[0:3]
    mid = x[3:6]
    a = 1.0 + m * (c - 1.0)
    b = m * s
    o_ref[...] = jnp.concatenate(
        [a * top - b * mid, b * top + a * mid, x[6:9]], axis=0)


@jax.jit
def _rotate(Rt, mrow, cs):
    return pl.pallas_call(
        _body,
        grid=(_GRID,),
        in_specs=[
            pl.BlockSpec(memory_space=pltpu.MemorySpace.SMEM),
            pl.BlockSpec((9, _BN), lambda i: (0, i)),
            pl.BlockSpec((1, _BN), lambda i: (0, i)),
        ],
        out_specs=pl.BlockSpec((9, _BN), lambda i: (0, i)),
        out_shape=jax.ShapeDtypeStruct((9, _N), jnp.float32),
    )(cs, Rt, mrow)


def kernel(R, mask, value):
    angle = jnp.float32(value)
    cs = jnp.stack([jnp.cos(angle), jnp.sin(angle)])
    Rt = jnp.transpose(R, (1, 2, 0)).reshape(9, _N)
    mrow = mask.astype(jnp.float32).reshape(1, _N)
    out = _rotate(Rt, mrow, cs)
    return jnp.transpose(out.reshape(3, 3, _N), (2, 0, 1))
